# baseline (speedup 1.0000x reference)
import jax
import jax.numpy as jnp
from jax.experimental import pallas as pl

BATCH = 16384
NUM_FEATURES = 26
EMB_DIM = 64


def _tc_body(x_ref, w_ref, out_ref, norm_ref):
    w = w_ref[...]
    e = jnp.exp(w)
    s = jnp.sum(e, axis=0, keepdims=True)
    n = e / s
    norm_ref[...] = n
    x = x_ref[...]
    out_ref[...] = jnp.sum(x * n[:, None, :], axis=0)


def _tc_pool(xt, gathered_t, block_b=1024):
    nb = BATCH // block_b
    out_shapes = (
        jax.ShapeDtypeStruct((EMB_DIM, BATCH), jnp.float32),
        jax.ShapeDtypeStruct((NUM_FEATURES, BATCH), jnp.float32),
    )
    return pl.pallas_call(
        _tc_body,
        grid=(nb,),
        in_specs=[
            pl.BlockSpec((NUM_FEATURES, EMB_DIM, block_b), lambda i: (0, 0, i)),
            pl.BlockSpec((NUM_FEATURES, block_b), lambda i: (0, i)),
        ],
        out_specs=(
            pl.BlockSpec((EMB_DIM, block_b), lambda i: (0, i)),
            pl.BlockSpec((NUM_FEATURES, block_b), lambda i: (0, i)),
        ),
        out_shape=out_shapes,
    )(xt, gathered_t)


def _tc_pool_chunk(xt, gathered_t, chunk, nchunks, block_b=1024):
    nb = BATCH // block_b // nchunks
    cb = BATCH // nchunks
    out_shapes = (
        jax.ShapeDtypeStruct((EMB_DIM, cb), jnp.float32),
        jax.ShapeDtypeStruct((NUM_FEATURES, cb), jnp.float32),
    )
    off = chunk * nb
    return pl.pallas_call(
        _tc_body,
        grid=(nb,),
        in_specs=[
            pl.BlockSpec((NUM_FEATURES, EMB_DIM, block_b),
                         lambda i: (0, 0, off + i)),
            pl.BlockSpec((NUM_FEATURES, block_b), lambda i: (0, i)),
        ],
        out_specs=(
            pl.BlockSpec((EMB_DIM, block_b), lambda i: (0, i)),
            pl.BlockSpec((NUM_FEATURES, block_b), lambda i: (0, i)),
        ),
        out_shape=out_shapes,
    )(xt, gathered_t)


@jax.jit
def kernel(inputs, item_indices, attention_weights):
    xt = jnp.transpose(inputs, (1, 2, 0))
    nchunks = 4
    cb = BATCH // nchunks
    outs, norms = [], []
    for c in range(nchunks):
        idx_c = jax.lax.slice(item_indices, (c * cb,), ((c + 1) * cb,))
        g = jnp.take(attention_weights, idx_c, axis=0)
        o, n = _tc_pool_chunk(xt, g.T, c, nchunks)
        outs.append(o)
        norms.append(n)
    out_t = jnp.concatenate(outs, axis=1)
    norm_t = jnp.concatenate(norms, axis=1)
    return out_t.T, norm_t.T[:, :, None]


# speedup vs baseline: 1.1492x; 1.1492x over previous
import jax
import jax.numpy as jnp
from jax.experimental import pallas as pl

BATCH = 16384
NUM_FEATURES = 26
EMB_DIM = 64


def _tc_body(x_ref, w_ref, out_ref, norm_ref):
    w = w_ref[...]
    e = jnp.exp(w)
    s = jnp.sum(e, axis=0, keepdims=True)
    n = e / s
    norm_ref[...] = n
    x = x_ref[...]
    out_ref[...] = jnp.sum(x * n[:, None, :], axis=0)


def _tc_pool(xt, gathered_t, block_b=2048):
    nb = BATCH // block_b
    out_shapes = (
        jax.ShapeDtypeStruct((EMB_DIM, BATCH), jnp.float32),
        jax.ShapeDtypeStruct((NUM_FEATURES, BATCH), jnp.float32),
    )
    return pl.pallas_call(
        _tc_body,
        grid=(nb,),
        in_specs=[
            pl.BlockSpec((NUM_FEATURES, EMB_DIM, block_b), lambda i: (0, 0, i)),
            pl.BlockSpec((NUM_FEATURES, block_b), lambda i: (0, i)),
        ],
        out_specs=(
            pl.BlockSpec((EMB_DIM, block_b), lambda i: (0, i)),
            pl.BlockSpec((NUM_FEATURES, block_b), lambda i: (0, i)),
        ),
        out_shape=out_shapes,
    )(xt, gathered_t)


@jax.jit
def kernel(inputs, item_indices, attention_weights):
    xt = jnp.transpose(inputs, (1, 2, 0))
    g = jnp.take(attention_weights, item_indices, axis=0)
    out_t, norm_t = _tc_pool(xt, g.T)
    return out_t.T, norm_t.T[:, :, None]
